# baseline (speedup 1.0000x reference)
import functools

import jax
import jax.numpy as jnp
from jax import lax
from jax.experimental import pallas as pl
from jax.experimental.pallas import tpu as pltpu
from jax.experimental.pallas import tpu_sc as plsc

N_LEAVES = 32
D = 256
LANES = 16
CHUNKS = D // LANES
B = 4
S = 32
GROUPS_PER_BATCH = 8
Q_PER_WORKER = S // GROUPS_PER_BATCH
LEVELS = 5


def _sc_body(q_hbm, k_hbm, v_hbm, vl_hbm, o_hbm, tk, tv, qv, ov, vlv, slots):
    nc = 2
    cid = lax.axis_index("c")
    sid = lax.axis_index("s")
    wid = sid * nc + cid
    b = wid // GROUPS_PER_BATCH
    g = wid % GROUPS_PER_BATCH
    q0 = g * Q_PER_WORKER

    zero_i = jnp.zeros((LANES,), jnp.int32)
    one_i = jnp.full((LANES,), 1, jnp.int32)
    zero_f = jnp.zeros((LANES,), jnp.float32)
    one_f = jnp.full((LANES,), 1.0, jnp.float32)
    half_f = jnp.float32(0.5)

    pltpu.sync_copy(vl_hbm, vlv)
    pltpu.sync_copy(k_hbm.at[b, pl.ds(1, 31)], tk.at[pl.ds(N_LEAVES, 31)])
    pltpu.sync_copy(k_hbm.at[b, pl.ds(31, 1)], tk.at[pl.ds(63, 1)])
    pltpu.sync_copy(v_hbm.at[b, pl.ds(1, 31)], tv.at[pl.ds(N_LEAVES, 31)])
    pltpu.sync_copy(v_hbm.at[b, pl.ds(31, 1)], tv.at[pl.ds(63, 1)])
    pltpu.sync_copy(q_hbm.at[b, pl.ds(q0, Q_PER_WORKER)], qv)

    nv = plsc.load_gather(vlv, [jnp.full((LANES,), b, jnp.int32)])
    n_s = nv[0].astype(jnp.float32)

    def leaf_mask(p):
        pf = jnp.asarray(p, jnp.int32).astype(jnp.float32)
        return jnp.where(pf < n_s, jnp.float32(1.0), jnp.float32(0.0))

    @plsc.parallel_loop(0, 16, unroll=4)
    def _l4_body(t):
        m0 = jnp.full((LANES,), leaf_mask(2 * t + 1), jnp.float32)
        m1 = jnp.full((LANES,), leaf_mask(2 * t + 2), jnp.float32)
        r0 = N_LEAVES + 2 * t
        r1 = r0 + 1
        w = 16 + t
        for c in range(CHUNKS):
            sl = pl.ds(c * LANES, LANES)
            tk[w, sl] = tk[r0, sl] * m0 + tk[r1, sl] * m1
            tv[w, sl] = tv[r0, sl] * m0 + tv[r1, sl] * m1

    for lo, hi in ((8, 16), (4, 8), (2, 4)):
        @plsc.parallel_loop(lo, hi, unroll=2)
        def _lvl_body(m):
            r0 = 2 * m
            r1 = r0 + 1
            for c in range(CHUNKS):
                sl = pl.ds(c * LANES, LANES)
                tk[m, sl] = tk[r0, sl] + tk[r1, sl]
                tv[m, sl] = tv[r0, sl] + tv[r1, sl]

    @plsc.parallel_loop(0, Q_PER_WORKER, unroll=2)
    def _q_body(qi):
        qiv = jnp.full((LANES,), qi, jnp.int32)

        @plsc.parallel_loop(0, CHUNKS, unroll=4)
        def _zero_ans(c):
            ov[qi, pl.ds(c * LANES, LANES)] = zero_f

        def level_body(t, idx):
            left = idx * 2
            right = left + 1
            is_leaf = left >= N_LEAVES
            ml = jnp.where(is_leaf, leaf_mask(left - 31), jnp.float32(1.0))
            mr = jnp.where(is_leaf, leaf_mask(right - 31), jnp.float32(1.0))

            @plsc.parallel_loop(0, CHUNKS, unroll=4, carry=(zero_f, zero_f))
            def accs(c, acc):
                accl, accr = acc
                sl = pl.ds(c * LANES, LANES)
                qcv = qv[qi, sl]
                accl = accl + qcv * tk[left, sl]
                accr = accr + qcv * tk[right, sl]
                return accl, accr

            accl = accs[0] * jnp.full((LANES,), ml, jnp.float32)
            accr = accs[1] * jnp.full((LANES,), mr, jnp.float32)
            slots[qi, pl.ds(0, LANES)] = zero_f
            plsc.addupdate_scatter(slots, [qiv, zero_i], accl)
            plsc.addupdate_scatter(slots, [qiv, one_i], accr)
            dots = slots[qi, pl.ds(0, LANES)]
            sg = one_f / (one_f + jnp.exp(-dots))
            ls = sg[0]
            rs = sg[1]
            cond = ls >= rs
            s = jnp.where(cond, rs, ls) * jnp.where(cond, mr, ml)
            sv = jnp.full((LANES,), s, jnp.float32)
            crow = jnp.where(cond, right, left)

            @plsc.parallel_loop(0, CHUNKS, unroll=4)
            def _ans(c):
                sl = pl.ds(c * LANES, LANES)
                ov[qi, sl] = ov[qi, sl] * half_f + sv * tv[crow, sl]

            return jnp.where(cond, left, right)

        lax.fori_loop(0, LEVELS, level_body, jnp.int32(1))

    pltpu.sync_copy(ov, o_hbm.at[b, pl.ds(q0, Q_PER_WORKER)])


@jax.jit
def _sc_call(queries, keys, values, vl_pad):
    mesh = plsc.VectorSubcoreMesh(core_axis_name="c", subcore_axis_name="s")
    f = functools.partial(
        pl.kernel,
        out_type=jax.ShapeDtypeStruct((B, S, D), jnp.float32),
        mesh=mesh,
        compiler_params=pltpu.CompilerParams(
            use_tc_tiling_on_sc=False, needs_layout_passes=False
        ),
        scratch_types=[
            pltpu.VMEM((2 * N_LEAVES, D), jnp.float32),
            pltpu.VMEM((2 * N_LEAVES, D), jnp.float32),
            pltpu.VMEM((Q_PER_WORKER, D), jnp.float32),
            pltpu.VMEM((Q_PER_WORKER, D), jnp.float32),
            pltpu.VMEM((LANES,), jnp.int32),
            pltpu.VMEM((Q_PER_WORKER, LANES), jnp.float32),
        ],
    )(_sc_body)
    return f(queries, keys, values, vl_pad)


def kernel(queries, keys, values, valid_lens):
    vl_pad = jnp.zeros((LANES,), jnp.int32).at[:B].set(valid_lens.astype(jnp.int32))
    return _sc_call(queries, keys, values, vl_pad)


# speedup vs baseline: 1.0094x; 1.0094x over previous
import functools

import jax
import jax.numpy as jnp
from jax import lax
from jax.experimental import pallas as pl
from jax.experimental.pallas import tpu as pltpu
from jax.experimental.pallas import tpu_sc as plsc

N_LEAVES = 32
D = 256
LANES = 16
CHUNKS = D // LANES
B = 4
S = 32
GROUPS_PER_BATCH = 8
Q_PER_WORKER = S // GROUPS_PER_BATCH
LEVELS = 5


def _sc_body(q_hbm, k_hbm, v_hbm, vl_hbm, o_hbm,
             l5k, l5v, l4k, l4v, l3k, l3v, l2k, l2v, l1k, l1v,
             qv, ov, vlv, slots):
    nc = 2
    cid = lax.axis_index("c")
    sid = lax.axis_index("s")
    wid = sid * nc + cid
    b = wid // GROUPS_PER_BATCH
    g = wid % GROUPS_PER_BATCH
    q0 = g * Q_PER_WORKER

    zero_i = jnp.zeros((LANES,), jnp.int32)
    one_i = jnp.full((LANES,), 1, jnp.int32)
    zero_f = jnp.zeros((LANES,), jnp.float32)
    one_f = jnp.full((LANES,), 1.0, jnp.float32)
    half_f = jnp.float32(0.5)

    pltpu.sync_copy(vl_hbm, vlv)
    pltpu.sync_copy(k_hbm.at[b, pl.ds(1, 31)], l5k.at[pl.ds(0, 31)])
    pltpu.sync_copy(k_hbm.at[b, pl.ds(31, 1)], l5k.at[pl.ds(31, 1)])
    pltpu.sync_copy(v_hbm.at[b, pl.ds(1, 31)], l5v.at[pl.ds(0, 31)])
    pltpu.sync_copy(v_hbm.at[b, pl.ds(31, 1)], l5v.at[pl.ds(31, 1)])
    pltpu.sync_copy(q_hbm.at[b, pl.ds(q0, Q_PER_WORKER)], qv)

    nv = plsc.load_gather(vlv, [jnp.full((LANES,), b, jnp.int32)])
    n_s = nv[0].astype(jnp.float32)

    def leaf_mask(p):
        pf = jnp.asarray(p, jnp.int32).astype(jnp.float32)
        return jnp.where(pf < n_s, jnp.float32(1.0), jnp.float32(0.0))

    @plsc.parallel_loop(0, 16, unroll=4)
    def _l4_body(t):
        m0 = jnp.full((LANES,), leaf_mask(2 * t + 1), jnp.float32)
        m1 = jnp.full((LANES,), leaf_mask(2 * t + 2), jnp.float32)
        r0 = 2 * t
        r1 = r0 + 1
        for c in range(CHUNKS):
            sl = pl.ds(c * LANES, LANES)
            l4k[t, sl] = l5k[r0, sl] * m0 + l5k[r1, sl] * m1
            l4v[t, sl] = l5v[r0, sl] * m0 + l5v[r1, sl] * m1

    @plsc.parallel_loop(0, 8, unroll=4)
    def _l3_body(t):
        r0 = 2 * t
        r1 = r0 + 1
        for c in range(CHUNKS):
            sl = pl.ds(c * LANES, LANES)
            l3k[t, sl] = l4k[r0, sl] + l4k[r1, sl]
            l3v[t, sl] = l4v[r0, sl] + l4v[r1, sl]

    for t in range(4):
        for c in range(CHUNKS):
            sl = pl.ds(c * LANES, LANES)
            l2k[t, sl] = l3k[2 * t, sl] + l3k[2 * t + 1, sl]
            l2v[t, sl] = l3v[2 * t, sl] + l3v[2 * t + 1, sl]
    for t in range(2):
        for c in range(CHUNKS):
            sl = pl.ds(c * LANES, LANES)
            l1k[t, sl] = l2k[2 * t, sl] + l2k[2 * t + 1, sl]
            l1v[t, sl] = l2v[2 * t, sl] + l2v[2 * t + 1, sl]

    ks_by_depth = [l1k, l2k, l3k, l4k, l5k]
    vs_by_depth = [l1v, l2v, l3v, l4v, l5v]

    @plsc.parallel_loop(0, Q_PER_WORKER, unroll=2)
    def _q_body(qi):
        qc = [qv[qi, pl.ds(c * LANES, LANES)] for c in range(CHUNKS)]
        idx = jnp.int32(1)
        ans = [zero_f for _ in range(CHUNKS)]
        qiv = jnp.full((LANES,), qi, jnp.int32)
        for t in range(LEVELS):
            kref = ks_by_depth[t]
            vref = vs_by_depth[t]
            base = 1 << (t + 1)
            left = idx * 2
            right = left + 1
            lrow = left - base
            rrow = lrow + 1
            pl_accl = [zero_f, zero_f]
            pl_accr = [zero_f, zero_f]
            for c in range(CHUNKS):
                sl = pl.ds(c * LANES, LANES)
                p = c & 1
                pl_accl[p] = pl_accl[p] + qc[c] * kref[lrow, sl]
                pl_accr[p] = pl_accr[p] + qc[c] * kref[rrow, sl]
            accl = pl_accl[0] + pl_accl[1]
            accr = pl_accr[0] + pl_accr[1]
            if t == LEVELS - 1:
                ml = leaf_mask(left - 31)
                mr = leaf_mask(right - 31)
                accl = accl * jnp.full((LANES,), ml, jnp.float32)
                accr = accr * jnp.full((LANES,), mr, jnp.float32)
            slots[qi, pl.ds(0, LANES)] = zero_f
            plsc.addupdate_scatter(slots, [qiv, zero_i], accl)
            plsc.addupdate_scatter(slots, [qiv, one_i], accr)
            dots = slots[qi, pl.ds(0, LANES)]
            sg = one_f / (one_f + jnp.exp(-dots))
            ls = sg[0]
            rs = sg[1]
            cond = ls >= rs
            s = jnp.where(cond, rs, ls)
            if t == LEVELS - 1:
                s = s * jnp.where(cond, mr, ml)
            sv = jnp.full((LANES,), s, jnp.float32)
            crow = jnp.where(cond, rrow, lrow)
            for c in range(CHUNKS):
                sl = pl.ds(c * LANES, LANES)
                ans[c] = ans[c] * half_f + sv * vref[crow, sl]
            idx = jnp.where(cond, left, right)
        for c in range(CHUNKS):
            ov[qi, pl.ds(c * LANES, LANES)] = ans[c]

    pltpu.sync_copy(ov, o_hbm.at[b, pl.ds(q0, Q_PER_WORKER)])


@jax.jit
def _sc_call(queries, keys, values, vl_pad):
    mesh = plsc.VectorSubcoreMesh(core_axis_name="c", subcore_axis_name="s")
    f = functools.partial(
        pl.kernel,
        out_type=jax.ShapeDtypeStruct((B, S, D), jnp.float32),
        mesh=mesh,
        compiler_params=pltpu.CompilerParams(
            use_tc_tiling_on_sc=False, needs_layout_passes=False
        ),
        scratch_types=[
            pltpu.VMEM((32, D), jnp.float32),
            pltpu.VMEM((32, D), jnp.float32),
            pltpu.VMEM((16, D), jnp.float32),
            pltpu.VMEM((16, D), jnp.float32),
            pltpu.VMEM((8, D), jnp.float32),
            pltpu.VMEM((8, D), jnp.float32),
            pltpu.VMEM((4, D), jnp.float32),
            pltpu.VMEM((4, D), jnp.float32),
            pltpu.VMEM((2, D), jnp.float32),
            pltpu.VMEM((2, D), jnp.float32),
            pltpu.VMEM((Q_PER_WORKER, D), jnp.float32),
            pltpu.VMEM((Q_PER_WORKER, D), jnp.float32),
            pltpu.VMEM((LANES,), jnp.int32),
            pltpu.VMEM((Q_PER_WORKER, LANES), jnp.float32),
        ],
    )(_sc_body)
    return f(queries, keys, values, vl_pad)


def kernel(queries, keys, values, valid_lens):
    vl_pad = jnp.zeros((LANES,), jnp.int32).at[:B].set(valid_lens.astype(jnp.int32))
    return _sc_call(queries, keys, values, vl_pad)
